# baseline (speedup 1.0000x reference)
import functools

import jax
import jax.numpy as jnp
from jax import lax
from jax.experimental import pallas as pl
from jax.experimental.pallas import tpu as pltpu
from jax.experimental.pallas import tpu_sc as plsc

LANES = 16
NUM_CORES = 2
NUM_SUBCORES = 16
NUM_WORKERS = NUM_CORES * NUM_SUBCORES
CH = 64
KREGS = CH // LANES
B = 4096
L = 200
BBLK = B // NUM_WORKERS
CHUNK = BBLK
TW = 128
NBUF = 4
STEPS = 64
EPS = 1e-5


def _rsqrt_scalar(x):
    i = lax.bitcast_convert_type(x, jnp.int32)
    y = lax.bitcast_convert_type(
        jnp.int32(0x5F3759DF) - jnp.right_shift(i, jnp.int32(1)), jnp.float32)
    for _ in range(3):
        y = y * (1.5 - 0.5 * x * y * y)
    return y


def _rsqrt_vec(x):
    i = plsc.bitcast(x, jnp.int32)
    magic = jnp.full((LANES,), 0x5F3759DF, jnp.int32)
    one = jnp.full((LANES,), 1, jnp.int32)
    y = plsc.bitcast(magic - lax.shift_right_logical(i, one), jnp.float32)
    for _ in range(3):
        y = y * (1.5 - 0.5 * x * y * y)
    return y


def _sc_body(tok_hbm, tid_hbm, table_hbm, ttable_hbm,
             gam_hbm, bet_hbm, tgam_hbm, tbet_hbm, out_hbm,
             idx_t, tid_t, rb0, rb1, st0, st1, st2, st3,
             tmp_v, gam_v, bet_v, tgam_v, tbet_v,
             sg0, sg1, so0, so1, so2, so3):
    rows_bufs = [rb0, rb1]
    stage_bufs = [st0, st1, st2, st3]
    sem_g = [sg0, sg1]
    sem_o = [so0, so1, so2, so3]
    wid = lax.axis_index("s") * NUM_CORES + lax.axis_index("c")
    iota = lax.iota(jnp.int32, LANES)

    pltpu.sync_copy(tok_hbm.at[:, pl.ds(wid * BBLK, BBLK)], idx_t)
    pltpu.sync_copy(tid_hbm.at[:, pl.ds(wid * BBLK, BBLK)], tid_t)
    pltpu.sync_copy(ttable_hbm, tmp_v)
    pltpu.sync_copy(gam_hbm, gam_v)
    pltpu.sync_copy(bet_hbm, bet_v)
    pltpu.sync_copy(tgam_hbm, tgam_v)
    pltpu.sync_copy(tbet_hbm, tbet_v)

    def setup_body(j, carry):
        rbase = (j * LANES + iota) * CH
        s = jnp.zeros((LANES,), jnp.float32)
        ss = jnp.zeros((LANES,), jnp.float32)
        for c in range(CH):
            x = plsc.load_gather(tmp_v, [rbase + c])
            s = s + x
            ss = ss + x * x
        mu = s * (1.0 / CH)
        var = ss * (1.0 / CH) - mu * mu
        r = _rsqrt_vec(var + EPS)
        for c in range(CH):
            cc = jnp.full((LANES,), c, jnp.int32)
            x = plsc.load_gather(tmp_v, [rbase + c])
            g = plsc.load_gather(tgam_v, [cc])
            b = plsc.load_gather(tbet_v, [cc]) + plsc.load_gather(bet_v, [cc])
            plsc.store_scatter(tmp_v, [rbase + c], (x - mu) * r * g + b)
        return carry

    lax.fori_loop(0, STEPS // LANES, setup_body, 0)

    gam = [gam_v[pl.ds(k * LANES, LANES)] for k in range(KREGS)]
    ihi = jnp.right_shift(iota, 3)
    iloc = jnp.bitwise_and(iota, 7) * BBLK

    def issue_gather(l, b):
        pltpu.async_copy(table_hbm.at[idx_t.at[l]], rows_bufs[b], sem_g[b])

    def wait_gather(l, b):
        pltpu.make_async_copy(
            table_hbm.at[idx_t.at[l]], rows_bufs[b], sem_g[b]).wait()

    def wait_out(b):
        for ch in range(8):
            pltpu.make_async_copy(
                stage_bufs[b].at[ch], out_hbm.at[0, ch, wid], sem_o[b]).wait()

    def compute(l, gb, sb):
        rows_v = rows_bufs[gb]
        stage_v = stage_bufs[sb]
        ls = jnp.full((LANES,), l, jnp.int32)

        @plsc.parallel_loop(0, CHUNK, unroll=2)
        def row_body(r):
            x = [rows_v[r, pl.ds(k * LANES, LANES)] for k in range(KREGS)]
            s4 = x[0] + x[1] + x[2] + x[3]
            q4 = x[0] * x[0] + x[1] * x[1] + x[2] * x[2] + x[3] * x[3]
            mu = jnp.sum(s4) * (1.0 / CH)
            var = jnp.sum(q4) * (1.0 / CH) - mu * mu
            rstd = _rsqrt_scalar(var + EPS)
            rs = jnp.full((LANES,), r, jnp.int32)
            tsplat = plsc.load_gather(tid_t, [ls, rs])
            tb = tsplat * CH + iota
            inner = iloc + r
            for k in range(KREGS):
                t = plsc.load_gather(tmp_v, [tb + (k * LANES)])
                y = (x[k] - mu) * rstd * gam[k] + t
                plsc.store_scatter(stage_v, [2 * k + ihi, inner], y)

        for ch in range(8):
            pltpu.async_copy(stage_v.at[ch], out_hbm.at[l, ch, wid], sem_o[sb])

    issue_gather(0, 0)

    def super_body(i, carry):
        for p in range(NBUF):
            c = i * NBUF + p

            @pl.when(c + 1 < L)
            def _():
                issue_gather(c + 1, (p + 1) % 2)

            wait_gather(c, p % 2)

            @pl.when(i >= 1)
            def _():
                wait_out(p)

            compute(c, p % 2, p)
        return carry

    lax.fori_loop(0, L // NBUF, super_body, 0)
    for p in range(NBUF):
        wait_out(p)


_mesh = plsc.VectorSubcoreMesh(core_axis_name="c", subcore_axis_name="s")

_sc_call = functools.partial(
    pl.kernel,
    mesh=_mesh,
    compiler_params=pltpu.CompilerParams(
        use_tc_tiling_on_sc=False, needs_layout_passes=False),
    out_type=jax.ShapeDtypeStruct((L, 8, NUM_WORKERS, 8 * BBLK), jnp.float32),
    scratch_types=[
        pltpu.VMEM((L, BBLK), jnp.int32),
        pltpu.VMEM((L, BBLK), jnp.int32),
        pltpu.VMEM((CHUNK, TW), jnp.float32),
        pltpu.VMEM((CHUNK, TW), jnp.float32),
        pltpu.VMEM((8, 8 * BBLK), jnp.float32),
        pltpu.VMEM((8, 8 * BBLK), jnp.float32),
        pltpu.VMEM((8, 8 * BBLK), jnp.float32),
        pltpu.VMEM((8, 8 * BBLK), jnp.float32),
        pltpu.VMEM((STEPS * CH,), jnp.float32),
        pltpu.VMEM((CH,), jnp.float32),
        pltpu.VMEM((CH,), jnp.float32),
        pltpu.VMEM((CH,), jnp.float32),
        pltpu.VMEM((CH,), jnp.float32),
        pltpu.SemaphoreType.DMA,
        pltpu.SemaphoreType.DMA,
        pltpu.SemaphoreType.DMA,
        pltpu.SemaphoreType.DMA,
        pltpu.SemaphoreType.DMA,
        pltpu.SemaphoreType.DMA,
    ],
)(_sc_body)


def kernel(tokens, time_step, token_table, temporal_table, token_gamma,
           token_beta, temporal_gamma, temporal_beta):
    table_pad = jnp.pad(token_table, ((0, 0), (0, TW - CH)))
    out4 = _sc_call(tokens.T, time_step.T, table_pad,
                    temporal_table.reshape(STEPS * CH),
                    token_gamma, token_beta, temporal_gamma, temporal_beta)
    out5 = out4.reshape(L, 8, NUM_WORKERS, 8, BBLK)
    return out5.transpose(2, 4, 0, 1, 3).reshape(B, L, CH)


# speedup vs baseline: 1.0870x; 1.0870x over previous
import functools

import jax
import jax.numpy as jnp
from jax import lax
from jax.experimental import pallas as pl
from jax.experimental.pallas import tpu as pltpu
from jax.experimental.pallas import tpu_sc as plsc

LANES = 16
NUM_CORES = 2
NUM_SUBCORES = 16
NUM_WORKERS = NUM_CORES * NUM_SUBCORES
CH = 64
KREGS = CH // LANES
B = 4096
L = 200
BBLK = B // NUM_WORKERS
CHUNK = BBLK
TW = 128
NBUF = 4
STEPS = 64
EPS = 1e-5


def _rsqrt_scalar(x):
    i = lax.bitcast_convert_type(x, jnp.int32)
    y = lax.bitcast_convert_type(
        jnp.int32(0x5F3759DF) - jnp.right_shift(i, jnp.int32(1)), jnp.float32)
    for _ in range(3):
        y = y * (1.5 - 0.5 * x * y * y)
    return y


def _rsqrt_vec(x):
    i = plsc.bitcast(x, jnp.int32)
    magic = jnp.full((LANES,), 0x5F3759DF, jnp.int32)
    one = jnp.full((LANES,), 1, jnp.int32)
    y = plsc.bitcast(magic - lax.shift_right_logical(i, one), jnp.float32)
    for _ in range(3):
        y = y * (1.5 - 0.5 * x * y * y)
    return y


def _sc_body(tok_hbm, tid_hbm, table_hbm, ttable_hbm,
             gam_hbm, bet_hbm, tgam_hbm, tbet_hbm, out_hbm,
             idx_t, tid_t, rb0, rb1, st0, st1, st2, st3,
             tmp_v, gam_v, bet_v, tgam_v, tbet_v,
             sg0, sg1, so0, so1, so2, so3):
    rows_bufs = [rb0, rb1]
    stage_bufs = [st0, st1, st2, st3]
    sem_g = [sg0, sg1]
    sem_o = [so0, so1, so2, so3]
    wid = lax.axis_index("s") * NUM_CORES + lax.axis_index("c")
    iota = lax.iota(jnp.int32, LANES)

    pltpu.sync_copy(tok_hbm.at[:, pl.ds(wid * BBLK, BBLK)], idx_t)
    pltpu.sync_copy(tid_hbm.at[:, pl.ds(wid * BBLK, BBLK)], tid_t)
    pltpu.sync_copy(ttable_hbm, tmp_v)
    pltpu.sync_copy(gam_hbm, gam_v)
    pltpu.sync_copy(bet_hbm, bet_v)
    pltpu.sync_copy(tgam_hbm, tgam_v)
    pltpu.sync_copy(tbet_hbm, tbet_v)

    def setup_body(j, carry):
        rbase = (j * LANES + iota) * CH
        s = jnp.zeros((LANES,), jnp.float32)
        ss = jnp.zeros((LANES,), jnp.float32)
        for c in range(CH):
            x = plsc.load_gather(tmp_v, [rbase + c])
            s = s + x
            ss = ss + x * x
        mu = s * (1.0 / CH)
        var = ss * (1.0 / CH) - mu * mu
        r = _rsqrt_vec(var + EPS)
        for c in range(CH):
            cc = jnp.full((LANES,), c, jnp.int32)
            x = plsc.load_gather(tmp_v, [rbase + c])
            g = plsc.load_gather(tgam_v, [cc])
            b = plsc.load_gather(tbet_v, [cc]) + plsc.load_gather(bet_v, [cc])
            plsc.store_scatter(tmp_v, [rbase + c], (x - mu) * r * g + b)
        return carry

    lax.fori_loop(0, STEPS // LANES, setup_body, 0)

    gam = [gam_v[pl.ds(k * LANES, LANES)] for k in range(KREGS)]

    def issue_gather(l, b):
        pltpu.async_copy(table_hbm.at[idx_t.at[l]], rows_bufs[b], sem_g[b])

    def wait_gather(l, b):
        pltpu.make_async_copy(
            table_hbm.at[idx_t.at[l]], rows_bufs[b], sem_g[b]).wait()

    def wait_out(b):
        pltpu.make_async_copy(
            stage_bufs[b], out_hbm.at[pl.ds(wid * BBLK, BBLK), 0],
            sem_o[b]).wait()

    def compute(l, gb, sb):
        rows_v = rows_bufs[gb]
        stage_v = stage_bufs[sb]
        ls = jnp.full((LANES,), l, jnp.int32)

        @plsc.parallel_loop(0, CHUNK, unroll=4)
        def row_body(r):
            x = [rows_v[r, pl.ds(k * LANES, LANES)] for k in range(KREGS)]
            s4 = x[0] + x[1] + x[2] + x[3]
            q4 = x[0] * x[0] + x[1] * x[1] + x[2] * x[2] + x[3] * x[3]
            mu = jnp.sum(s4) * (1.0 / CH)
            var = jnp.sum(q4) * (1.0 / CH) - mu * mu
            rstd = _rsqrt_scalar(var + EPS)
            rs = jnp.full((LANES,), r, jnp.int32)
            tsplat = plsc.load_gather(tid_t, [ls, rs])
            tb = tsplat * CH + iota
            for k in range(KREGS):
                t = plsc.load_gather(tmp_v, [tb + (k * LANES)])
                y = (x[k] - mu) * rstd * gam[k] + t
                stage_v[r, pl.ds(k * LANES, LANES)] = y

        pltpu.async_copy(stage_v, out_hbm.at[pl.ds(wid * BBLK, BBLK), l],
                         sem_o[sb])

    issue_gather(0, 0)

    def super_body(i, carry):
        for p in range(NBUF):
            c = i * NBUF + p

            @pl.when(c + 1 < L)
            def _():
                issue_gather(c + 1, (p + 1) % 2)

            wait_gather(c, p % 2)

            @pl.when(i >= 1)
            def _():
                wait_out(p)

            compute(c, p % 2, p)
        return carry

    lax.fori_loop(0, L // NBUF, super_body, 0)
    for p in range(NBUF):
        wait_out(p)


_mesh = plsc.VectorSubcoreMesh(core_axis_name="c", subcore_axis_name="s")

_sc_call = functools.partial(
    pl.kernel,
    mesh=_mesh,
    compiler_params=pltpu.CompilerParams(
        use_tc_tiling_on_sc=False, needs_layout_passes=False),
    out_type=jax.ShapeDtypeStruct((B, L, CH), jnp.float32),
    scratch_types=[
        pltpu.VMEM((L, BBLK), jnp.int32),
        pltpu.VMEM((L, BBLK), jnp.int32),
        pltpu.VMEM((CHUNK, TW), jnp.float32),
        pltpu.VMEM((CHUNK, TW), jnp.float32),
        pltpu.VMEM((CHUNK, CH), jnp.float32),
        pltpu.VMEM((CHUNK, CH), jnp.float32),
        pltpu.VMEM((CHUNK, CH), jnp.float32),
        pltpu.VMEM((CHUNK, CH), jnp.float32),
        pltpu.VMEM((STEPS * CH,), jnp.float32),
        pltpu.VMEM((CH,), jnp.float32),
        pltpu.VMEM((CH,), jnp.float32),
        pltpu.VMEM((CH,), jnp.float32),
        pltpu.VMEM((CH,), jnp.float32),
        pltpu.SemaphoreType.DMA,
        pltpu.SemaphoreType.DMA,
        pltpu.SemaphoreType.DMA,
        pltpu.SemaphoreType.DMA,
        pltpu.SemaphoreType.DMA,
        pltpu.SemaphoreType.DMA,
    ],
)(_sc_body)


def kernel(tokens, time_step, token_table, temporal_table, token_gamma,
           token_beta, temporal_gamma, temporal_beta):
    table_pad = jnp.pad(token_table, ((0, 0), (0, TW - CH)))
    return _sc_call(tokens.T, time_step.T, table_pad,
                    temporal_table.reshape(STEPS * CH),
                    token_gamma, token_beta, temporal_gamma, temporal_beta)
